# baseline (speedup 1.0000x reference)
import jax
import jax.numpy as jnp
from jax.experimental import pallas as pl
from jax.experimental.pallas import tpu as pltpu

_B, _H, _S = 32, 1024, 576
_ND, _ED, _K = 4, 256, 1024
_COMMIT = 0.25
_BS = _B * _S
_TB = 512
_NSTEPS = _BS // _TB
_NELEMS = _B * _H * _S

_PREC_DIST = jax.lax.Precision.DEFAULT


def _vq_body(zrow_ref, zcol_ref, emb_ref, ehi_ref, elo_ref, elo2_ref,
             enc_ref, commit_ref, loss_ref, acc_ref, e2_ref):
    i = pl.program_id(0)

    @pl.when(i == 0)
    def _init():
        acc_ref[0, 0] = 0.0
        e2_ref[...] = jnp.sum(emb_ref[...] * emb_ref[...], axis=1)

    total = jnp.float32(0.0)
    for d in range(_ND):
        zd = zrow_ref[:, d * _ED:(d + 1) * _ED]
        emb = emb_ref[d]
        e2 = e2_ref[d]
        z2 = jnp.sum(zd * zd, axis=1)
        g = jax.lax.dot_general(
            zd, emb, (((1,), (0,)), ((), ())),
            preferred_element_type=jnp.float32,
            precision=_PREC_DIST)
        scores = (z2[:, None] + e2[None, :]) - 2.0 * g
        idx = jnp.argmin(scores, axis=1)
        eq = (jax.lax.broadcasted_iota(jnp.int32, (_K, _TB), 0)
              == idx[None, :])
        onehot = eq.astype(jnp.float32).astype(jnp.bfloat16)
        dims = (((1,), (0,)), ((), ()))
        enc_ref[d] = jax.lax.dot_general(
            ehi_ref[d], onehot, dims, preferred_element_type=jnp.float32)
        enc_ref[d] += jax.lax.dot_general(
            elo_ref[d], onehot, dims, preferred_element_type=jnp.float32)
        enc_ref[d] += jax.lax.dot_general(
            elo2_ref[d], onehot, dims, preferred_element_type=jnp.float32)
        enc = enc_ref[d]
        diff = zcol_ref[d] - enc
        total += jnp.sum(diff * diff)

    acc_ref[0, 0] += total

    @pl.when(i == _NSTEPS - 1)
    def _fin():
        c = acc_ref[0, 0] / jnp.float32(_NELEMS)
        commit_ref[0, 0] = c
        loss_ref[0, 0] = jnp.float32(_COMMIT) * c


def kernel(inputs, embeddings):
    zrow = inputs.reshape(_BS, _ND * _ED)
    zcol = inputs.reshape(_ND, _ED, _BS)
    ehi = embeddings.astype(jnp.bfloat16)
    r1 = embeddings - ehi.astype(jnp.float32)
    elo = r1.astype(jnp.bfloat16)
    elo2 = (r1 - elo.astype(jnp.float32)).astype(jnp.bfloat16)
    full = pl.BlockSpec((_ND, _ED, _K), lambda i: (0, 0, 0))
    enc, commit, loss = pl.pallas_call(
        _vq_body,
        grid=(_NSTEPS,),
        in_specs=[
            pl.BlockSpec((_TB, _ND * _ED), lambda i: (i, 0)),
            pl.BlockSpec((_ND, _ED, _TB), lambda i: (0, 0, i)),
            full, full, full, full,
        ],
        out_specs=[
            pl.BlockSpec((_ND, _ED, _TB), lambda i: (0, 0, i)),
            pl.BlockSpec(memory_space=pltpu.SMEM),
            pl.BlockSpec(memory_space=pltpu.SMEM),
        ],
        out_shape=[
            jax.ShapeDtypeStruct((_ND, _ED, _BS), jnp.float32),
            jax.ShapeDtypeStruct((1, 1), jnp.float32),
            jax.ShapeDtypeStruct((1, 1), jnp.float32),
        ],
        scratch_shapes=[pltpu.SMEM((1, 1), jnp.float32),
                        pltpu.VMEM((_ND, _K), jnp.float32)],
    )(zrow, zcol, embeddings, ehi, elo, elo2)
    output = enc.reshape(_B, _H, _S)
    commit_loss = commit[0, 0]
    kl = jnp.array(0)
    return (output, loss[0, 0], commit_loss, kl)


# speedup vs baseline: 1.0136x; 1.0136x over previous
import jax
import jax.numpy as jnp
from jax.experimental import pallas as pl
from jax.experimental.pallas import tpu as pltpu

_B, _H, _S = 32, 1024, 576
_ND, _ED, _K = 4, 256, 1024
_COMMIT = 0.25
_BS = _B * _S
_TB = 512
_NSTEPS = _BS // _TB
_NELEMS = _B * _H * _S

_PREC_DIST = jax.lax.Precision.DEFAULT


def _vq_body(zrow_ref, zcol_ref, emb_ref, ehi_ref, elo_ref, elo2_ref,
             enc_ref, commit_ref, loss_ref, acc_ref, e2_ref):
    i = pl.program_id(0)

    @pl.when(i == 0)
    def _init():
        acc_ref[0, 0] = 0.0
        e2_ref[...] = jnp.sum(emb_ref[...] * emb_ref[...], axis=1)

    total = jnp.float32(0.0)
    for d in range(_ND):
        zd = zrow_ref[:, d * _ED:(d + 1) * _ED]
        emb = emb_ref[d]
        e2 = e2_ref[d]
        z2 = jnp.sum(zd * zd, axis=1)
        g = jax.lax.dot_general(
            zd, emb, (((1,), (0,)), ((), ())),
            preferred_element_type=jnp.float32,
            precision=_PREC_DIST)
        scores = (z2[:, None] + e2[None, :]) - 2.0 * g
        idx = jnp.argmin(scores, axis=1)
        eq = (jax.lax.broadcasted_iota(jnp.int32, (_K, _TB), 0)
              == idx[None, :])
        onehot = eq.astype(jnp.float32).astype(jnp.bfloat16)
        dims = (((1,), (0,)), ((), ()))
        enc_ref[d] = jax.lax.dot_general(
            ehi_ref[d], onehot, dims, preferred_element_type=jnp.float32)
        enc_ref[d] += jax.lax.dot_general(
            elo_ref[d], onehot, dims, preferred_element_type=jnp.float32)
        enc_ref[d] += jax.lax.dot_general(
            elo2_ref[d], onehot, dims, preferred_element_type=jnp.float32)
        enc = enc_ref[d]
        diff = zcol_ref[d] - enc
        total += jnp.sum(diff * diff)

    acc_ref[0, 0] += total

    @pl.when(i == _NSTEPS - 1)
    def _fin():
        c = acc_ref[0, 0] / jnp.float32(_NELEMS)
        commit_ref[0, 0] = c
        loss_ref[0, 0] = jnp.float32(_COMMIT) * c


def kernel(inputs, embeddings):
    zrow = inputs.reshape(_BS, _ND * _ED)
    zcol = inputs.reshape(_ND, _ED, _BS)
    ehi = embeddings.astype(jnp.bfloat16)
    r1 = embeddings - jax.lax.optimization_barrier(ehi).astype(jnp.float32)
    elo = r1.astype(jnp.bfloat16)
    elo2 = (r1 - jax.lax.optimization_barrier(elo).astype(jnp.float32)
            ).astype(jnp.bfloat16)
    full = pl.BlockSpec((_ND, _ED, _K), lambda i: (0, 0, 0))
    enc, commit, loss = pl.pallas_call(
        _vq_body,
        grid=(_NSTEPS,),
        in_specs=[
            pl.BlockSpec((_TB, _ND * _ED), lambda i: (i, 0)),
            pl.BlockSpec((_ND, _ED, _TB), lambda i: (0, 0, i)),
            full, full, full, full,
        ],
        out_specs=[
            pl.BlockSpec((_ND, _ED, _TB), lambda i: (0, 0, i)),
            pl.BlockSpec(memory_space=pltpu.SMEM),
            pl.BlockSpec(memory_space=pltpu.SMEM),
        ],
        out_shape=[
            jax.ShapeDtypeStruct((_ND, _ED, _BS), jnp.float32),
            jax.ShapeDtypeStruct((1, 1), jnp.float32),
            jax.ShapeDtypeStruct((1, 1), jnp.float32),
        ],
        scratch_shapes=[pltpu.SMEM((1, 1), jnp.float32),
                        pltpu.VMEM((_ND, _K), jnp.float32)],
    )(zrow, zcol, embeddings, ehi, elo, elo2)
    output = enc.reshape(_B, _H, _S)
    commit_loss = commit[0, 0]
    kl = jnp.array(0)
    return (output, loss[0, 0], commit_loss, kl)
